# baseline (speedup 1.0000x reference)
import functools

import jax
import jax.numpy as jnp
from jax.experimental import pallas as pl
from jax.experimental.pallas import tpu as pltpu

_CDIM0 = (((0,), (0,)), ((), ()))
_BF16 = jnp.bfloat16


def _pick_block(n, cands):
    for c in cands:
        if n % c == 0:
            return c
    return n


def _sweep1_body(a_ref, x_ref, w1_ref, deg_ref, ab_ref, ynm_ref, *, num_r):
    r = pl.program_id(0)

    @pl.when(r == 0)
    def _init():
        deg_ref[...] = jnp.zeros_like(deg_ref)

    a = a_ref[...]
    s = jnp.sum(a, axis=0, keepdims=True)
    deg_ref[...] += jnp.broadcast_to(s, deg_ref.shape)
    ab_ref[...] = a.astype(jnp.int8)

    @pl.when(r == num_r - 1)
    def _tail():
        xw = jax.lax.dot_general(x_ref[...], w1_ref[...],
                                 (((1,), (0,)), ((), ())),
                                 preferred_element_type=jnp.float32)
        ynm_ref[...] = xw.astype(_BF16)
        deg_ref[...] = jax.lax.rsqrt(deg_ref[...] + 1.0)


def _l1_body(ynm_ref, ab_ref, dnm_ref, xt_ref, w1t_ref, dfm_ref, b1_ref,
             pa_ref, w2tp_ref, y2fm_ref, y2nm_ref, z_ref, yfm_ref, *, num_r):
    r = pl.program_id(0)

    @pl.when(r == 0)
    def _init():
        z_ref[...] = jnp.zeros_like(z_ref)
        y1 = jax.lax.dot_general(w1t_ref[...], xt_ref[...],
                                 (((1,), (0,)), ((), ())),
                                 preferred_element_type=jnp.float32)
        yfm_ref[...] = y1 * dfm_ref[0:1, :]

    y = (ynm_ref[...].astype(jnp.float32) * dnm_ref[...]).astype(_BF16)
    z_ref[...] += jax.lax.dot_general(y, ab_ref[...].astype(_BF16), _CDIM0,
                                      preferred_element_type=jnp.float32)

    @pl.when(r == num_r - 1)
    def _tail():
        d = dfm_ref[0:1, :]
        pre = (z_ref[...] + yfm_ref[...]) * d + b1_ref[...]
        alpha = pa_ref[0, 0]
        h = jnp.where(pre >= 0.0, pre, alpha * pre)
        y2 = jax.lax.dot_general(w2tp_ref[...], h,
                                 (((1,), (0,)), ((), ())),
                                 preferred_element_type=jnp.float32)
        y2 = y2 * d
        y2fm_ref[...] = y2
        y2nm_ref[...] = y2.T.astype(_BF16)


def _l2_body(y2nm_ref, ab_ref, y2fm_ref, dfm_ref, b2_ref, lt_ref, z_ref,
             *, num_r, num_classes):
    r = pl.program_id(0)

    @pl.when(r == 0)
    def _init():
        z_ref[...] = jnp.zeros_like(z_ref)

    z_ref[...] += jax.lax.dot_general(y2nm_ref[...],
                                      ab_ref[...].astype(_BF16), _CDIM0,
                                      preferred_element_type=jnp.float32)

    @pl.when(r == num_r - 1)
    def _tail():
        logits = (z_ref[...] + y2fm_ref[...]) * dfm_ref[0:1, :] + b2_ref[...]
        valid = (jax.lax.broadcasted_iota(jnp.int32, logits.shape, 0)
                 < num_classes)
        m = jnp.max(jnp.where(valid, logits, jnp.float32(-1e30)),
                    axis=0, keepdims=True)
        e = jnp.where(valid, jnp.exp(logits - m), 0.0)
        lse = jnp.log(jnp.sum(e, axis=0, keepdims=True)) + m
        lt_ref[...] = (logits - lse).T


def kernel(x, A, W1, b1, prelu_a, W2, b2):
    n, d = x.shape
    h = W1.shape[1]
    c = W2.shape[1]
    cp = 16
    f32 = jnp.float32

    bc = _pick_block(n, [400, 80, 16])
    bm = _pick_block(n, [1000, 400, 80])
    nc = n // bc
    nm = n // bm

    x_t = x.T
    w1t = W1.T
    w2tp = jnp.zeros((cp, h), f32).at[:c].set(W2.T)
    b1c = b1.reshape(h, 1)
    b2c = jnp.zeros((cp, 1), f32).at[:c, 0].set(b2)
    pa = jnp.full((8, 128), prelu_a, f32)

    deg8, ab, ynm = pl.pallas_call(
        functools.partial(_sweep1_body, num_r=nc),
        grid=(nc,),
        in_specs=[
            pl.BlockSpec((bc, n), lambda r: (r, 0)),
            pl.BlockSpec((n, d), lambda r: (0, 0)),
            pl.BlockSpec((d, h), lambda r: (0, 0)),
        ],
        out_specs=[
            pl.BlockSpec((8, n), lambda r: (0, 0)),
            pl.BlockSpec((bc, n), lambda r: (r, 0)),
            pl.BlockSpec((n, h), lambda r: (0, 0)),
        ],
        out_shape=[
            jax.ShapeDtypeStruct((8, n), f32),
            jax.ShapeDtypeStruct((n, n), jnp.int8),
            jax.ShapeDtypeStruct((n, h), _BF16),
        ],
    )(A, x, W1)
    dfm = deg8
    dnm = dfm[0:1, :].T

    y2fm, y2nm = pl.pallas_call(
        functools.partial(_l1_body, num_r=nm),
        grid=(nm,),
        in_specs=[
            pl.BlockSpec((bm, h), lambda r: (r, 0)),
            pl.BlockSpec((bm, n), lambda r: (r, 0)),
            pl.BlockSpec((bm, 1), lambda r: (r, 0)),
            pl.BlockSpec((d, n), lambda r: (0, 0)),
            pl.BlockSpec((h, d), lambda r: (0, 0)),
            pl.BlockSpec((8, n), lambda r: (0, 0)),
            pl.BlockSpec((h, 1), lambda r: (0, 0)),
            pl.BlockSpec((8, 128), lambda r: (0, 0)),
            pl.BlockSpec((cp, h), lambda r: (0, 0)),
        ],
        out_specs=[
            pl.BlockSpec((cp, n), lambda r: (0, 0)),
            pl.BlockSpec((n, cp), lambda r: (0, 0)),
        ],
        out_shape=[
            jax.ShapeDtypeStruct((cp, n), f32),
            jax.ShapeDtypeStruct((n, cp), _BF16),
        ],
        scratch_shapes=[
            pltpu.VMEM((h, n), f32),
            pltpu.VMEM((h, n), f32),
        ],
    )(ynm, ab, dnm, x_t, w1t, dfm, b1c, pa, w2tp)

    lt = pl.pallas_call(
        functools.partial(_l2_body, num_r=nm, num_classes=c),
        grid=(nm,),
        in_specs=[
            pl.BlockSpec((bm, cp), lambda r: (r, 0)),
            pl.BlockSpec((bm, n), lambda r: (r, 0)),
            pl.BlockSpec((cp, n), lambda r: (0, 0)),
            pl.BlockSpec((8, n), lambda r: (0, 0)),
            pl.BlockSpec((cp, 1), lambda r: (0, 0)),
        ],
        out_specs=pl.BlockSpec((n, cp), lambda r: (0, 0)),
        out_shape=jax.ShapeDtypeStruct((n, cp), f32),
        scratch_shapes=[pltpu.VMEM((cp, n), f32)],
    )(y2nm, ab, y2fm, dfm, b2c)

    return lt[:, :c]


# speedup vs baseline: 1.0153x; 1.0153x over previous
import functools

import jax
import jax.numpy as jnp
from jax.experimental import pallas as pl
from jax.experimental.pallas import tpu as pltpu

_CDIM0 = (((0,), (0,)), ((), ()))
_BF16 = jnp.bfloat16


def _pick_block(n, cands):
    for c in cands:
        if n % c == 0:
            return c
    return n


def _sweep1_body(a_ref, x_ref, w1_ref, deg_ref, ab_ref, ynm_ref, *, num_r):
    r = pl.program_id(0)

    @pl.when(r == 0)
    def _init():
        deg_ref[...] = jnp.zeros_like(deg_ref)

    a = a_ref[...]
    s = jnp.sum(a, axis=0, keepdims=True)
    deg_ref[...] += jnp.broadcast_to(s, deg_ref.shape)
    ab_ref[...] = a.astype(jnp.int8)

    @pl.when(r == num_r - 1)
    def _tail():
        xw = jax.lax.dot_general(x_ref[...], w1_ref[...],
                                 (((1,), (0,)), ((), ())),
                                 preferred_element_type=jnp.float32)
        ynm_ref[...] = xw.astype(_BF16)
        deg_ref[...] = jax.lax.rsqrt(deg_ref[...] + 1.0)


def _l1_body(ynm_ref, ab_ref, dnm_ref, xt_ref, w1t_ref, dfm_ref, b1_ref,
             pa_ref, w2tp_ref, y2fm_ref, y2nm_ref, z_ref, yfm_ref, *, num_r):
    r = pl.program_id(0)

    @pl.when(r == 0)
    def _init():
        z_ref[...] = jnp.zeros_like(z_ref)
        y1 = jax.lax.dot_general(w1t_ref[...], xt_ref[...],
                                 (((1,), (0,)), ((), ())),
                                 preferred_element_type=jnp.float32)
        yfm_ref[...] = y1 * dfm_ref[0:1, :]

    y = (ynm_ref[...].astype(jnp.float32) * dnm_ref[...]).astype(_BF16)
    z_ref[...] += jax.lax.dot_general(y, ab_ref[...].astype(_BF16), _CDIM0,
                                      preferred_element_type=jnp.float32)

    @pl.when(r == num_r - 1)
    def _tail():
        d = dfm_ref[0:1, :]
        pre = (z_ref[...] + yfm_ref[...]) * d + b1_ref[...]
        alpha = pa_ref[0, 0]
        h = jnp.where(pre >= 0.0, pre, alpha * pre)
        y2 = jax.lax.dot_general(w2tp_ref[...], h,
                                 (((1,), (0,)), ((), ())),
                                 preferred_element_type=jnp.float32)
        y2 = y2 * d
        y2fm_ref[...] = y2
        y2nm_ref[...] = y2.T.astype(_BF16)


def _l2_body(y2nm_ref, ab_ref, y2fm_ref, dfm_ref, b2_ref, lt_ref, z_ref,
             *, num_r, num_classes):
    r = pl.program_id(0)

    @pl.when(r == 0)
    def _init():
        z_ref[...] = jnp.zeros_like(z_ref)

    z_ref[...] += jax.lax.dot_general(y2nm_ref[...],
                                      ab_ref[...].astype(_BF16), _CDIM0,
                                      preferred_element_type=jnp.float32)

    @pl.when(r == num_r - 1)
    def _tail():
        logits = (z_ref[...] + y2fm_ref[...]) * dfm_ref[0:1, :] + b2_ref[...]
        valid = (jax.lax.broadcasted_iota(jnp.int32, logits.shape, 0)
                 < num_classes)
        m = jnp.max(jnp.where(valid, logits, jnp.float32(-1e30)),
                    axis=0, keepdims=True)
        e = jnp.where(valid, jnp.exp(logits - m), 0.0)
        lse = jnp.log(jnp.sum(e, axis=0, keepdims=True)) + m
        lt_ref[...] = logits - lse


def kernel(x, A, W1, b1, prelu_a, W2, b2):
    n, d = x.shape
    h = W1.shape[1]
    c = W2.shape[1]
    cp = 16
    f32 = jnp.float32

    bc = _pick_block(n, [400, 80, 16])
    bm = _pick_block(n, [1000, 400, 80])
    nc = n // bc
    nm = n // bm

    x_t = x.T
    w1t = W1.T
    w2tp = jnp.zeros((cp, h), f32).at[:c].set(W2.T)
    b1c = b1.reshape(h, 1)
    b2c = jnp.zeros((cp, 1), f32).at[:c, 0].set(b2)
    pa = jnp.full((8, 128), prelu_a, f32)

    deg8, ab, ynm = pl.pallas_call(
        functools.partial(_sweep1_body, num_r=nc),
        grid=(nc,),
        in_specs=[
            pl.BlockSpec((bc, n), lambda r: (r, 0)),
            pl.BlockSpec((n, d), lambda r: (0, 0)),
            pl.BlockSpec((d, h), lambda r: (0, 0)),
        ],
        out_specs=[
            pl.BlockSpec((8, n), lambda r: (0, 0)),
            pl.BlockSpec((bc, n), lambda r: (r, 0)),
            pl.BlockSpec((n, h), lambda r: (0, 0)),
        ],
        out_shape=[
            jax.ShapeDtypeStruct((8, n), f32),
            jax.ShapeDtypeStruct((n, n), jnp.int8),
            jax.ShapeDtypeStruct((n, h), _BF16),
        ],
    )(A, x, W1)
    dfm = deg8
    dnm = dfm[0:1, :].T

    y2fm, y2nm = pl.pallas_call(
        functools.partial(_l1_body, num_r=nm),
        grid=(nm,),
        in_specs=[
            pl.BlockSpec((bm, h), lambda r: (r, 0)),
            pl.BlockSpec((bm, n), lambda r: (r, 0)),
            pl.BlockSpec((bm, 1), lambda r: (r, 0)),
            pl.BlockSpec((d, n), lambda r: (0, 0)),
            pl.BlockSpec((h, d), lambda r: (0, 0)),
            pl.BlockSpec((8, n), lambda r: (0, 0)),
            pl.BlockSpec((h, 1), lambda r: (0, 0)),
            pl.BlockSpec((8, 128), lambda r: (0, 0)),
            pl.BlockSpec((cp, h), lambda r: (0, 0)),
        ],
        out_specs=[
            pl.BlockSpec((cp, n), lambda r: (0, 0)),
            pl.BlockSpec((n, cp), lambda r: (0, 0)),
        ],
        out_shape=[
            jax.ShapeDtypeStruct((cp, n), f32),
            jax.ShapeDtypeStruct((n, cp), _BF16),
        ],
        scratch_shapes=[
            pltpu.VMEM((h, n), f32),
            pltpu.VMEM((h, n), f32),
        ],
    )(ynm, ab, dnm, x_t, w1t, dfm, b1c, pa, w2tp)

    lt = pl.pallas_call(
        functools.partial(_l2_body, num_r=nm, num_classes=c),
        grid=(nm,),
        in_specs=[
            pl.BlockSpec((bm, cp), lambda r: (r, 0)),
            pl.BlockSpec((bm, n), lambda r: (r, 0)),
            pl.BlockSpec((cp, n), lambda r: (0, 0)),
            pl.BlockSpec((8, n), lambda r: (0, 0)),
            pl.BlockSpec((cp, 1), lambda r: (0, 0)),
        ],
        out_specs=pl.BlockSpec((cp, n), lambda r: (0, 0)),
        out_shape=jax.ShapeDtypeStruct((cp, n), f32),
        scratch_shapes=[pltpu.VMEM((cp, n), f32)],
    )(y2nm, ab, y2fm, dfm, b2c)

    return lt[:c, :].T
